# Initial kernel scaffold; baseline (speedup 1.0000x reference)
import functools

import jax
import jax.numpy as jnp
import numpy as np
from jax import lax
from jax.experimental import pallas as pl
from jax.experimental.pallas import tpu as pltpu
from jax.experimental.pallas import tpu_sc as plsc

_SQ6 = float(np.sqrt(1.0 + 1e-6))
_SQ5 = float(np.sqrt(1.0 + 1e-5))


def _lrelu(h, s=0.2):
    return jnp.where(h >= 0, h, s * h)


@functools.lru_cache(maxsize=None)
def _make_sc_gather(V, C, B):
    nw = 32
    bpw = B // nw
    mesh = plsc.VectorSubcoreMesh(core_axis_name="c", subcore_axis_name="s")

    @functools.partial(
        pl.kernel,
        out_type=jax.ShapeDtypeStruct((B, C), jnp.float32),
        mesh=mesh,
        scratch_types=[
            pltpu.VMEM((bpw,), jnp.int32),
            pltpu.VMEM((bpw, C), jnp.float32),
            pltpu.SemaphoreType.DMA,
        ],
    )
    def gather_k(table_hbm, idx_hbm, out_hbm, idx_v, rows_v, sem):
        wid = lax.axis_index("s") * 2 + lax.axis_index("c")
        base = wid * bpw
        pltpu.sync_copy(idx_hbm.at[pl.ds(base, bpw)], idx_v)
        pltpu.async_copy(table_hbm.at[idx_v], rows_v, sem).wait()
        pltpu.sync_copy(rows_v, out_hbm.at[pl.ds(base, bpw)])

    return gather_k


def _sc_gather(table, idx):
    V, C = table.shape
    (B,) = idx.shape
    return _make_sc_gather(V, C, B)(table, idx)


def _knn(qpos, pos, BQ, K=16):
    Qp = qpos.shape[0]
    N = pos.shape[0]

    def body(q_ref, p_ref, o_ref):
        q = q_ref[...]
        p = p_ref[...]
        qn = jnp.sum(q * q, axis=1, keepdims=True)
        pn = jnp.sum(p * p, axis=1)[None, :]
        d2 = qn + pn - 2.0 * lax.dot_general(
            q, p, (((1,), (1,)), ((), ())), preferred_element_type=jnp.float32)
        iota = lax.broadcasted_iota(jnp.int32, (BQ, N), 1)
        cols = []
        for _ in range(K):
            m = jnp.min(d2, axis=1, keepdims=True)
            am = jnp.min(jnp.where(d2 == m, iota, N), axis=1, keepdims=True)
            cols.append(am)
            d2 = jnp.where(iota == am, jnp.inf, d2)
        o_ref[...] = jnp.concatenate(cols, axis=1)

    return pl.pallas_call(
        body,
        grid=(Qp // BQ,),
        in_specs=[
            pl.BlockSpec((BQ, 3), lambda i: (i, 0)),
            pl.BlockSpec((N, 3), lambda i: (0, 0)),
        ],
        out_specs=pl.BlockSpec((BQ, K), lambda i: (i, 0)),
        out_shape=jax.ShapeDtypeStruct((Qp, K), jnp.int32),
    )(qpos, pos)


def _nn_idx(qpos, pos, BQ):
    Qp = qpos.shape[0]
    N = pos.shape[0]

    def body(q_ref, p_ref, o_ref):
        q = q_ref[...]
        p = p_ref[...]
        qn = jnp.sum(q * q, axis=1, keepdims=True)
        pn = jnp.sum(p * p, axis=1)[None, :]
        d2 = qn + pn - 2.0 * lax.dot_general(
            q, p, (((1,), (1,)), ((), ())), preferred_element_type=jnp.float32)
        iota = lax.broadcasted_iota(jnp.int32, (BQ, N), 1)
        m = jnp.min(d2, axis=1, keepdims=True)
        am = jnp.min(jnp.where(d2 == m, iota, N), axis=1, keepdims=True)
        o_ref[...] = am

    return pl.pallas_call(
        body,
        grid=(Qp // BQ,),
        in_specs=[
            pl.BlockSpec((BQ, 3), lambda i: (i, 0)),
            pl.BlockSpec((N, 3), lambda i: (0, 0)),
        ],
        out_specs=pl.BlockSpec((BQ, 1), lambda i: (i, 0)),
        out_shape=jax.ShapeDtypeStruct((Qp, 1), jnp.int32),
    )(qpos, pos)


def _pre_level1(pos, x, p):
    N = pos.shape[0]
    fc0, short, mlp1 = p['fc0'], p['b1']['short'], p['b1']['mlp1']

    def body(pos_ref, x_ref, w0, b0, g0, be0, ws, bs, gs, bes, w1, b1,
             h0_ref, sc_ref, tab_ref):
        f = jnp.concatenate([pos_ref[...], x_ref[...]], axis=1)
        h0 = jnp.dot(f, w0[...], preferred_element_type=jnp.float32) + b0[...]
        h0 = g0[...] * h0 / _SQ6 + be0[...]
        h0_ref[...] = h0
        sc = jnp.dot(h0, ws[...], preferred_element_type=jnp.float32) + bs[...]
        sc_ref[...] = gs[...] * sc / _SQ6 + bes[...]
        h = _lrelu(jnp.dot(h0, w1[...], preferred_element_type=jnp.float32) + b1[...])
        tab_ref[...] = jnp.concatenate(
            [pos_ref[...], h, jnp.zeros((N, 9), jnp.float32)], axis=1)

    return pl.pallas_call(
        body,
        out_shape=[
            jax.ShapeDtypeStruct((N, 16), jnp.float32),
            jax.ShapeDtypeStruct((N, 32), jnp.float32),
            jax.ShapeDtypeStruct((N, 16), jnp.float32),
        ],
    )(pos, x, fc0['W'], fc0['b'], fc0['g'], fc0['be'],
      short['W'], short['b'], short['g'], short['be'], mlp1['W'], mlp1['b'])


def _pre_level(f, pos, p, Ct):
    N, din = f.shape
    short, mlp1 = p['short'], p['mlp1']
    d4 = mlp1['W'].shape[1]
    padw = Ct - 3 - d4

    def body(f_ref, pos_ref, ws, bs, gs, bes, w1, b1, sc_ref, tab_ref):
        f_ = f_ref[...]
        sc = jnp.dot(f_, ws[...], preferred_element_type=jnp.float32) + bs[...]
        sc_ref[...] = gs[...] * sc / _SQ6 + bes[...]
        h = _lrelu(jnp.dot(f_, w1[...], preferred_element_type=jnp.float32) + b1[...])
        tab_ref[...] = jnp.concatenate(
            [pos_ref[...], h, jnp.zeros((N, padw), jnp.float32)], axis=1)

    return pl.pallas_call(
        body,
        out_shape=[
            jax.ShapeDtypeStruct((N, short['W'].shape[1]), jnp.float32),
            jax.ShapeDtypeStruct((N, Ct), jnp.float32),
        ],
    )(f, pos, short['W'], short['b'], short['g'], short['be'],
      mlp1['W'], mlp1['b'])


def _lfa_stage(gpos, gx, xoff, posi_rep, p, BE):
    E = gpos.shape[0]
    Cp = gpos.shape[1]
    Cx = gx.shape[1]
    enc, attn, post = p['enc'], p['attn'], p['post']
    xw = enc['W'].shape[1]
    d = attn['W'].shape[0]
    BQ = BE // 16
    Qp = E // 16

    def body(gp_ref, gx_ref, pi_ref, ew, eb, eg, ebe, aw, pw, pb, pg, pbe,
             o_ref):
        posj = gp_ref[:, 0:3]
        xj = gx_ref[:, xoff:xoff + xw]
        posi = pi_ref[...]
        dist = posj - posi
        euc = jnp.sum(jnp.sqrt(dist * dist), axis=1, keepdims=True)
        rel = jnp.concatenate([posi, posj, dist, euc], axis=1)
        lse = jnp.dot(rel, ew[...], preferred_element_type=jnp.float32) + eb[...]
        lse = _lrelu(eg[...] * lse / _SQ6 + ebe[...])
        local = jnp.concatenate([xj, lse], axis=1)
        logits = jnp.dot(local, aw[...], preferred_element_type=jnp.float32)
        mx = jnp.max(logits, axis=1, keepdims=True)
        ex = jnp.exp(logits - mx)
        attn_w = ex / jnp.sum(ex, axis=1, keepdims=True)
        msg = attn_w * local
        r = lax.broadcasted_iota(jnp.int32, (BQ, BE), 0)
        c = lax.broadcasted_iota(jnp.int32, (BQ, BE), 1)
        A = ((c >= r * 16) & (c < r * 16 + 16)).astype(jnp.float32)
        agg = jnp.dot(A, msg, preferred_element_type=jnp.float32)
        out = jnp.dot(agg, pw[...], preferred_element_type=jnp.float32) + pb[...]
        o_ref[...] = _lrelu(pg[...] * out / _SQ6 + pbe[...])

    return pl.pallas_call(
        body,
        grid=(E // BE,),
        in_specs=[
            pl.BlockSpec((BE, Cp), lambda i: (i, 0)),
            pl.BlockSpec((BE, Cx), lambda i: (i, 0)),
            pl.BlockSpec((BE, 3), lambda i: (i, 0)),
            pl.BlockSpec(enc['W'].shape, lambda i: (0, 0)),
            pl.BlockSpec(enc['b'].shape, lambda i: (0,)),
            pl.BlockSpec(enc['g'].shape, lambda i: (0,)),
            pl.BlockSpec(enc['be'].shape, lambda i: (0,)),
            pl.BlockSpec(attn['W'].shape, lambda i: (0, 0)),
            pl.BlockSpec(post['W'].shape, lambda i: (0, 0)),
            pl.BlockSpec(post['b'].shape, lambda i: (0,)),
            pl.BlockSpec(post['g'].shape, lambda i: (0,)),
            pl.BlockSpec(post['be'].shape, lambda i: (0,)),
        ],
        out_specs=pl.BlockSpec((BQ, d), lambda i: (i, 0)),
        out_shape=jax.ShapeDtypeStruct((Qp, d), jnp.float32),
    )(gpos, gx, posi_rep, enc['W'], enc['b'], enc['g'], enc['be'],
      attn['W'], post['W'], post['b'], post['g'], post['be'])


def _post_block(h, sc, p):
    N, d = h.shape
    d2o = p['W'].shape[1]

    def body(h_ref, sc_ref, w, b, g, be, o_ref):
        o = jnp.dot(h_ref[...], w[...], preferred_element_type=jnp.float32) + b[...]
        o = g[...] * o / _SQ6 + be[...]
        o = o + sc_ref[...]
        o_ref[...] = jnp.where(o >= 0, o, 0.01 * o)

    return pl.pallas_call(
        body,
        out_shape=jax.ShapeDtypeStruct((N, d2o), jnp.float32),
    )(h, sc, p['W'], p['b'], p['g'], p['be'])


def _fp_layer(a, b, p):
    N = a.shape[0]
    dout = p['W'].shape[1]

    def body(a_ref, b_ref, w, bb, g, be, o_ref):
        f = jnp.concatenate([a_ref[...], b_ref[...]], axis=1)
        h = jnp.dot(f, w[...], preferred_element_type=jnp.float32) + bb[...]
        h = g[...] * h / _SQ6 + be[...]
        o_ref[...] = _lrelu(h)

    return pl.pallas_call(
        body,
        out_shape=jax.ShapeDtypeStruct((N, dout), jnp.float32),
    )(a, b, p['W'], p['b'], p['g'], p['be'])


def _top_layer(a, p):
    N = a.shape[0]
    dout = p['W'].shape[1]

    def body(a_ref, w, bb, g, be, o_ref):
        h = jnp.dot(a_ref[...], w[...], preferred_element_type=jnp.float32) + bb[...]
        h = g[...] * h / _SQ5 + be[...]
        o_ref[...] = _lrelu(h)

    return pl.pallas_call(
        body,
        out_shape=jax.ShapeDtypeStruct((N, dout), jnp.float32),
    )(a, p['W'], p['b'], p['g'], p['be'])


def _final_chain(g, h0, params):
    N = g.shape[0]
    fp1, m2a, m2b, fce = params['fp1'], params['m2a'], params['m2b'], params['fc_end']

    def body(g_ref, h0_ref, w1, b1, g1, be1, wa, ba, ga, bea,
             wb, bb, gb, beb, we, be_, o_ref):
        f = jnp.concatenate([g_ref[...], h0_ref[...]], axis=1)
        h = jnp.dot(f, w1[...], preferred_element_type=jnp.float32) + b1[...]
        h = _lrelu(g1[...] * h / _SQ6 + be1[...])
        h = jnp.dot(h, wa[...], preferred_element_type=jnp.float32) + ba[...]
        h = _lrelu(ga[...] * h / _SQ6 + bea[...])
        h = jnp.dot(h, wb[...], preferred_element_type=jnp.float32) + bb[...]
        h = _lrelu(gb[...] * h / _SQ6 + beb[...])
        o_ref[...] = jnp.dot(h, we[...], preferred_element_type=jnp.float32) + be_[...]

    return pl.pallas_call(
        body,
        out_shape=jax.ShapeDtypeStruct((N, 13), jnp.float32),
    )(g, h0, fp1['W'], fp1['b'], fp1['g'], fp1['be'],
      m2a['W'], m2a['b'], m2a['g'], m2a['be'],
      m2b['W'], m2b['b'], m2b['g'], m2b['be'], fce['W'], fce['b'])


def _pad_rows(a, n, val=0.0):
    if a.shape[0] == n:
        return a
    return jnp.concatenate(
        [a, jnp.full((n - a.shape[0],) + a.shape[1:], val, a.dtype)], axis=0)


def _pad_cols(a, c):
    if a.shape[1] == c:
        return a
    return jnp.concatenate(
        [a, jnp.zeros((a.shape[0], c - a.shape[1]), a.dtype)], axis=1)


def _encoder_block(xf, posl, p, Qp, Ep, BQ, BE, Ct, C2):
    N = xf.shape[0]
    Q = (N + 3) // 4
    sc, table1 = _pre_level(xf, posl, p, Ct)
    qpos = _pad_rows(posl[::4], Qp)
    nn = _knn(qpos, posl, BQ)
    src = nn.reshape(-1).astype(jnp.int32)
    posi_rep = _pad_rows(jnp.repeat(posl[:Q], 16, axis=0), Ep)
    g1 = _sc_gather(table1, src)
    h1 = _lfa_stage(g1, g1, 3, posi_rep, p['lfa1'], BE)
    tab2 = _pad_cols(_pad_rows(h1[:Q], N), C2)
    g2 = _sc_gather(tab2, src)
    h2 = _lfa_stage(g1, g2, 0, posi_rep, p['lfa2'], BE)
    return _post_block(_pad_rows(h2[:Q], N), sc, p['mlp2'])


def _interp_gather(table, qpos, pxp, Bi, BQ):
    idx = _nn_idx(qpos, pxp, BQ).reshape(-1)
    idx = _pad_rows(idx.astype(jnp.int32), Bi)
    return _sc_gather(table, idx)


def kernel(pos, x, batch, params):
    del batch
    N1 = pos.shape[0]
    h0, sc1, table1 = _pre_level1(pos, x, params)
    b1 = params['b1']
    Q1, Qp1, Ep1 = 2500, 2560, 40960
    qpos1 = _pad_rows(pos[::4], Qp1)
    nn1 = _knn(qpos1, pos, 256)
    src1 = nn1.reshape(-1).astype(jnp.int32)
    posi1 = _pad_rows(jnp.repeat(pos[:Q1], 16, axis=0), Ep1)
    g1 = _sc_gather(table1, src1)
    h11 = _lfa_stage(g1, g1, 3, posi1, b1['lfa1'], 2048)
    tab2 = _pad_cols(_pad_rows(h11[:Q1], N1), 16)
    g2 = _sc_gather(tab2, src1)
    h12 = _lfa_stage(g1, g2, 0, posi1, b1['lfa2'], 2048)
    x1 = _post_block(_pad_rows(h12[:Q1], N1), sc1, b1['mlp2'])

    x1s, pos1 = x1[::4], pos[::4]
    x2 = _encoder_block(x1s, pos1, params['b2'],
                        Qp=640, Ep=10240, BQ=640, BE=2048, Ct=32, C2=32)
    x2s, pos2 = x2[::4], pos1[::4]
    x3 = _encoder_block(x2s, pos2, params['b3'],
                        Qp=160, Ep=2560, BQ=160, BE=2560, Ct=48, C2=64)
    x3s, pos3 = x3[::4], pos2[::4]
    x4 = _encoder_block(x3s, pos3, params['b4'],
                        Qp=48, Ep=768, BQ=48, BE=768, Ct=80, C2=128)
    x4s, pos4 = x4[::4], pos3[::4]

    xm = _top_layer(x4s, params['top'])
    pos4p = _pad_rows(pos4, 48, 1e6)
    pos3q = _pad_rows(pos3, 160)
    i4 = _interp_gather(xm, pos3q, pos4p, 256, 160)[:157]
    f4 = _fp_layer(i4, x3s, params['fp4'])

    pos3p = _pad_rows(pos3, 160, 1e6)
    pos2q = _pad_rows(pos2, 640)
    i3 = _interp_gather(f4, pos2q, pos3p, 768, 640)[:625]
    f3 = _fp_layer(i3, x2s, params['fp3'])

    posq = _pad_rows(pos, 10240)
    i2 = _interp_gather(f3, posq, pos2, 10240, 512)[:N1]
    f2 = _fp_layer(i2, x1, params['fp2'])

    i1 = _interp_gather(f2, posq, pos, 10240, 256)[:N1]
    return _final_chain(i1, h0, params)


# speedup vs baseline: 3.2377x; 3.2377x over previous
import functools

import jax
import jax.numpy as jnp
import numpy as np
from jax import lax
from jax.experimental import pallas as pl
from jax.experimental.pallas import tpu as pltpu
from jax.experimental.pallas import tpu_sc as plsc

_SQ6 = float(np.sqrt(1.0 + 1e-6))
_SQ5 = float(np.sqrt(1.0 + 1e-5))


def _lrelu(h, s=0.2):
    return jnp.where(h >= 0, h, s * h)


def _r2(v):
    return v.reshape(1, -1)


@functools.lru_cache(maxsize=None)
def _make_sc_gather(V, C, B):
    nw = 32
    bpw = B // nw
    ch = bpw
    while ch * C * 4 > 393216:
        ch //= 2
    nchunk = bpw // ch
    mesh = plsc.VectorSubcoreMesh(core_axis_name="c", subcore_axis_name="s")

    @functools.partial(
        pl.kernel,
        out_type=jax.ShapeDtypeStruct((B, C), jnp.float32),
        mesh=mesh,
        scratch_types=[
            pltpu.VMEM((ch,), jnp.int32),
            pltpu.VMEM((ch, C), jnp.float32),
            pltpu.SemaphoreType.DMA,
        ],
    )
    def gather_k(table_hbm, idx_hbm, out_hbm, idx_v, rows_v, sem):
        wid = lax.axis_index("s") * 2 + lax.axis_index("c")
        base = wid * bpw
        for j in range(nchunk):
            off = base + j * ch
            pltpu.sync_copy(idx_hbm.at[pl.ds(off, ch)], idx_v)
            pltpu.async_copy(table_hbm.at[idx_v], rows_v, sem).wait()
            pltpu.sync_copy(rows_v, out_hbm.at[pl.ds(off, ch)])

    return gather_k


def _sc_gather(table, idx):
    V, C = table.shape
    (B,) = idx.shape
    return _make_sc_gather(V, C, B)(table, idx)


def _knn(qpos, pos, BQ, K=16):
    Qp = qpos.shape[0]
    N = pos.shape[0]

    def body(q_ref, p_ref, o_ref):
        q = q_ref[...]
        p = p_ref[...]
        qn = jnp.sum(q * q, axis=1, keepdims=True)
        pn = jnp.sum(p * p, axis=1)[None, :]
        d2 = qn + pn - 2.0 * lax.dot_general(
            q, p, (((1,), (1,)), ((), ())), preferred_element_type=jnp.float32)
        iota = lax.broadcasted_iota(jnp.int32, (BQ, N), 1)
        cols = []
        for _ in range(K):
            m = jnp.min(d2, axis=1, keepdims=True)
            am = jnp.min(jnp.where(d2 == m, iota, N), axis=1, keepdims=True)
            cols.append(am)
            d2 = jnp.where(iota == am, jnp.inf, d2)
        o_ref[...] = jnp.concatenate(cols, axis=1)

    return pl.pallas_call(
        body,
        grid=(Qp // BQ,),
        in_specs=[
            pl.BlockSpec((BQ, 3), lambda i: (i, 0)),
            pl.BlockSpec((N, 3), lambda i: (0, 0)),
        ],
        out_specs=pl.BlockSpec((BQ, K), lambda i: (i, 0)),
        out_shape=jax.ShapeDtypeStruct((Qp, K), jnp.int32),
    )(qpos, pos)


def _nn_idx(qpos, pos, BQ):
    Qp = qpos.shape[0]
    N = pos.shape[0]

    def body(q_ref, p_ref, o_ref):
        q = q_ref[...]
        p = p_ref[...]
        qn = jnp.sum(q * q, axis=1, keepdims=True)
        pn = jnp.sum(p * p, axis=1)[None, :]
        d2 = qn + pn - 2.0 * lax.dot_general(
            q, p, (((1,), (1,)), ((), ())), preferred_element_type=jnp.float32)
        iota = lax.broadcasted_iota(jnp.int32, (BQ, N), 1)
        m = jnp.min(d2, axis=1, keepdims=True)
        am = jnp.min(jnp.where(d2 == m, iota, N), axis=1, keepdims=True)
        o_ref[...] = am

    return pl.pallas_call(
        body,
        grid=(Qp // BQ,),
        in_specs=[
            pl.BlockSpec((BQ, 3), lambda i: (i, 0)),
            pl.BlockSpec((N, 3), lambda i: (0, 0)),
        ],
        out_specs=pl.BlockSpec((BQ, 1), lambda i: (i, 0)),
        out_shape=jax.ShapeDtypeStruct((Qp, 1), jnp.int32),
    )(qpos, pos)


def _pre_level1(pos, x, p):
    N = pos.shape[0]
    fc0, short, mlp1 = p['fc0'], p['b1']['short'], p['b1']['mlp1']

    def body(pos_ref, x_ref, w0, b0, g0, be0, ws, bs, gs, bes, w1, b1,
             h0_ref, sc_ref, tab_ref):
        f = jnp.concatenate([pos_ref[...], x_ref[...]], axis=1)
        h0 = jnp.dot(f, w0[...], preferred_element_type=jnp.float32) + b0[...]
        h0 = g0[...] * h0 / _SQ6 + be0[...]
        h0_ref[...] = h0
        sc = jnp.dot(h0, ws[...], preferred_element_type=jnp.float32) + bs[...]
        sc_ref[...] = gs[...] * sc / _SQ6 + bes[...]
        h = _lrelu(jnp.dot(h0, w1[...], preferred_element_type=jnp.float32) + b1[...])
        tab_ref[...] = jnp.concatenate(
            [pos_ref[...], h, jnp.zeros((N, 121), jnp.float32)], axis=1)

    return pl.pallas_call(
        body,
        out_shape=[
            jax.ShapeDtypeStruct((N, 16), jnp.float32),
            jax.ShapeDtypeStruct((N, 32), jnp.float32),
            jax.ShapeDtypeStruct((N, 128), jnp.float32),
        ],
    )(pos, x, fc0['W'], _r2(fc0['b']), _r2(fc0['g']), _r2(fc0['be']),
      short['W'], _r2(short['b']), _r2(short['g']), _r2(short['be']),
      mlp1['W'], _r2(mlp1['b']))


def _pre_level(f, pos, p, Ct=128):
    N, din = f.shape
    short, mlp1 = p['short'], p['mlp1']
    d4 = mlp1['W'].shape[1]
    padw = Ct - 3 - d4

    def body(f_ref, pos_ref, ws, bs, gs, bes, w1, b1, sc_ref, tab_ref):
        f_ = f_ref[...]
        sc = jnp.dot(f_, ws[...], preferred_element_type=jnp.float32) + bs[...]
        sc_ref[...] = gs[...] * sc / _SQ6 + bes[...]
        h = _lrelu(jnp.dot(f_, w1[...], preferred_element_type=jnp.float32) + b1[...])
        tab_ref[...] = jnp.concatenate(
            [pos_ref[...], h, jnp.zeros((N, padw), jnp.float32)], axis=1)

    return pl.pallas_call(
        body,
        out_shape=[
            jax.ShapeDtypeStruct((N, short['W'].shape[1]), jnp.float32),
            jax.ShapeDtypeStruct((N, Ct), jnp.float32),
        ],
    )(f, pos, short['W'], _r2(short['b']), _r2(short['g']), _r2(short['be']),
      mlp1['W'], _r2(mlp1['b']))


def _lfa_stage(gpos, gx, xoff, posi_rep, p, BE):
    E = gpos.shape[0]
    Cp = gpos.shape[1]
    Cx = gx.shape[1]
    enc, attn, post = p['enc'], p['attn'], p['post']
    xw = enc['W'].shape[1]
    d = attn['W'].shape[0]
    BQ = BE // 16
    Qp = E // 16

    def body(gp_ref, gx_ref, pi_ref, ew, eb, eg, ebe, aw, pw, pb, pg, pbe,
             o_ref):
        posj = gp_ref[:, 0:3]
        xj = gx_ref[:, xoff:xoff + xw]
        posi = pi_ref[...]
        dist = posj - posi
        euc = jnp.sum(jnp.sqrt(dist * dist), axis=1, keepdims=True)
        rel = jnp.concatenate([posi, posj, dist, euc], axis=1)
        lse = jnp.dot(rel, ew[...], preferred_element_type=jnp.float32) + eb[...]
        lse = _lrelu(eg[...] * lse / _SQ6 + ebe[...])
        local = jnp.concatenate([xj, lse], axis=1)
        logits = jnp.dot(local, aw[...], preferred_element_type=jnp.float32)
        mx = jnp.max(logits, axis=1, keepdims=True)
        ex = jnp.exp(logits - mx)
        attn_w = ex / jnp.sum(ex, axis=1, keepdims=True)
        msg = attn_w * local
        r = lax.broadcasted_iota(jnp.int32, (BQ, BE), 0)
        c = lax.broadcasted_iota(jnp.int32, (BQ, BE), 1)
        A = ((c >= r * 16) & (c < r * 16 + 16)).astype(jnp.float32)
        agg = jnp.dot(A, msg, preferred_element_type=jnp.float32)
        out = jnp.dot(agg, pw[...], preferred_element_type=jnp.float32) + pb[...]
        o_ref[...] = _lrelu(pg[...] * out / _SQ6 + pbe[...])

    return pl.pallas_call(
        body,
        grid=(E // BE,),
        in_specs=[
            pl.BlockSpec((BE, Cp), lambda i: (i, 0)),
            pl.BlockSpec((BE, Cx), lambda i: (i, 0)),
            pl.BlockSpec((BE, 3), lambda i: (i, 0)),
            pl.BlockSpec(enc['W'].shape, lambda i: (0, 0)),
            pl.BlockSpec((1, xw), lambda i: (0, 0)),
            pl.BlockSpec((1, xw), lambda i: (0, 0)),
            pl.BlockSpec((1, xw), lambda i: (0, 0)),
            pl.BlockSpec(attn['W'].shape, lambda i: (0, 0)),
            pl.BlockSpec(post['W'].shape, lambda i: (0, 0)),
            pl.BlockSpec((1, d), lambda i: (0, 0)),
            pl.BlockSpec((1, d), lambda i: (0, 0)),
            pl.BlockSpec((1, d), lambda i: (0, 0)),
        ],
        out_specs=pl.BlockSpec((BQ, d), lambda i: (i, 0)),
        out_shape=jax.ShapeDtypeStruct((Qp, d), jnp.float32),
    )(gpos, gx, posi_rep, enc['W'], _r2(enc['b']), _r2(enc['g']), _r2(enc['be']),
      attn['W'], post['W'], _r2(post['b']), _r2(post['g']), _r2(post['be']))


def _post_block(h, sc, p):
    N, d = h.shape
    d2o = p['W'].shape[1]

    def body(h_ref, sc_ref, w, b, g, be, o_ref):
        o = jnp.dot(h_ref[...], w[...], preferred_element_type=jnp.float32) + b[...]
        o = g[...] * o / _SQ6 + be[...]
        o = o + sc_ref[...]
        o_ref[...] = jnp.where(o >= 0, o, 0.01 * o)

    return pl.pallas_call(
        body,
        out_shape=jax.ShapeDtypeStruct((N, d2o), jnp.float32),
    )(h, sc, p['W'], _r2(p['b']), _r2(p['g']), _r2(p['be']))


def _fp_layer(a, b, p):
    N = a.shape[0]
    dout = p['W'].shape[1]

    def body(a_ref, b_ref, w, bb, g, be, o_ref):
        f = jnp.concatenate([a_ref[...], b_ref[...]], axis=1)
        h = jnp.dot(f, w[...], preferred_element_type=jnp.float32) + bb[...]
        h = g[...] * h / _SQ6 + be[...]
        o_ref[...] = _lrelu(h)

    return pl.pallas_call(
        body,
        out_shape=jax.ShapeDtypeStruct((N, dout), jnp.float32),
    )(a, b, p['W'], _r2(p['b']), _r2(p['g']), _r2(p['be']))


def _top_layer(a, p):
    N = a.shape[0]
    dout = p['W'].shape[1]

    def body(a_ref, w, bb, g, be, o_ref):
        h = jnp.dot(a_ref[...], w[...], preferred_element_type=jnp.float32) + bb[...]
        h = g[...] * h / _SQ5 + be[...]
        o_ref[...] = _lrelu(h)

    return pl.pallas_call(
        body,
        out_shape=jax.ShapeDtypeStruct((N, dout), jnp.float32),
    )(a, p['W'], _r2(p['b']), _r2(p['g']), _r2(p['be']))


def _final_chain(g, h0, params):
    N = g.shape[0]
    fp1, m2a, m2b, fce = params['fp1'], params['m2a'], params['m2b'], params['fc_end']

    def body(g_ref, h0_ref, w1, b1, g1, be1, wa, ba, ga, bea,
             wb, bb, gb, beb, we, be_, o_ref):
        f = jnp.concatenate([g_ref[...], h0_ref[...]], axis=1)
        h = jnp.dot(f, w1[...], preferred_element_type=jnp.float32) + b1[...]
        h = _lrelu(g1[...] * h / _SQ6 + be1[...])
        h = jnp.dot(h, wa[...], preferred_element_type=jnp.float32) + ba[...]
        h = _lrelu(ga[...] * h / _SQ6 + bea[...])
        h = jnp.dot(h, wb[...], preferred_element_type=jnp.float32) + bb[...]
        h = _lrelu(gb[...] * h / _SQ6 + beb[...])
        o_ref[...] = jnp.dot(h, we[...], preferred_element_type=jnp.float32) + be_[...]

    return pl.pallas_call(
        body,
        out_shape=jax.ShapeDtypeStruct((N, 13), jnp.float32),
    )(g, h0, fp1['W'], _r2(fp1['b']), _r2(fp1['g']), _r2(fp1['be']),
      m2a['W'], _r2(m2a['b']), _r2(m2a['g']), _r2(m2a['be']),
      m2b['W'], _r2(m2b['b']), _r2(m2b['g']), _r2(m2b['be']),
      fce['W'], _r2(fce['b']))


def _pad_rows(a, n, val=0.0):
    if a.shape[0] == n:
        return a
    return jnp.concatenate(
        [a, jnp.full((n - a.shape[0],) + a.shape[1:], val, a.dtype)], axis=0)


def _pad_cols(a, c):
    if a.shape[1] == c:
        return a
    return jnp.concatenate(
        [a, jnp.zeros((a.shape[0], c - a.shape[1]), a.dtype)], axis=1)


def _encoder_block(xf, posl, p, Qp, Ep, BQ, BE):
    N = xf.shape[0]
    Q = (N + 3) // 4
    sc, table1 = _pre_level(xf, posl, p)
    qpos = _pad_rows(posl[::4], Qp)
    nn = _knn(qpos, posl, BQ)
    src = nn.reshape(-1).astype(jnp.int32)
    posi_rep = _pad_rows(jnp.repeat(posl[:Q], 16, axis=0), Ep)
    g1 = _sc_gather(table1, src)
    h1 = _lfa_stage(g1, g1, 3, posi_rep, p['lfa1'], BE)
    tab2 = _pad_cols(_pad_rows(h1[:Q], N), 128)
    g2 = _sc_gather(tab2, src)
    h2 = _lfa_stage(g1, g2, 0, posi_rep, p['lfa2'], BE)
    return _post_block(_pad_rows(h2[:Q], N), sc, p['mlp2'])


def _interp_gather(table, qpos, pxp, Bi, BQ):
    idx = _nn_idx(qpos, pxp, BQ).reshape(-1)
    idx = _pad_rows(idx.astype(jnp.int32), Bi)
    return _sc_gather(table, idx)


def kernel(pos, x, batch, params):
    del batch
    N1 = pos.shape[0]
    h0, sc1, table1 = _pre_level1(pos, x, params)
    b1 = params['b1']
    Q1, Qp1, Ep1 = 2500, 2560, 40960
    qpos1 = _pad_rows(pos[::4], Qp1)
    nn1 = _knn(qpos1, pos, 256)
    src1 = nn1.reshape(-1).astype(jnp.int32)
    posi1 = _pad_rows(jnp.repeat(pos[:Q1], 16, axis=0), Ep1)
    g1 = _sc_gather(table1, src1)
    h11 = _lfa_stage(g1, g1, 3, posi1, b1['lfa1'], 2048)
    tab2 = _pad_cols(_pad_rows(h11[:Q1], N1), 128)
    g2 = _sc_gather(tab2, src1)
    h12 = _lfa_stage(g1, g2, 0, posi1, b1['lfa2'], 2048)
    x1 = _post_block(_pad_rows(h12[:Q1], N1), sc1, b1['mlp2'])

    x1s, pos1 = x1[::4], pos[::4]
    x2 = _encoder_block(x1s, pos1, params['b2'],
                        Qp=640, Ep=10240, BQ=640, BE=2048)
    x2s, pos2 = x2[::4], pos1[::4]
    x3 = _encoder_block(x2s, pos2, params['b3'],
                        Qp=160, Ep=2560, BQ=160, BE=2560)
    x3s, pos3 = x3[::4], pos2[::4]
    x4 = _encoder_block(x3s, pos3, params['b4'],
                        Qp=48, Ep=768, BQ=48, BE=768)
    x4s, pos4 = x4[::4], pos3[::4]

    xm = _top_layer(x4s, params['top'])
    pos4p = _pad_rows(pos4, 48, 1e6)
    pos3q = _pad_rows(pos3, 160)
    i4 = _interp_gather(xm, pos3q, pos4p, 256, 160)[:157]
    f4 = _fp_layer(i4, x3s, params['fp4'])

    pos3p = _pad_rows(pos3, 160, 1e6)
    pos2q = _pad_rows(pos2, 640)
    i3 = _interp_gather(f4, pos2q, pos3p, 768, 640)[:625]
    f3 = _fp_layer(i3, x2s, params['fp3'])

    posq = _pad_rows(pos, 10240)
    i2 = _interp_gather(f3, posq, pos2, 10240, 512)[:N1]
    f2 = _fp_layer(i2, x1, params['fp2'])

    i1 = _interp_gather(_pad_cols(f2, 128), posq, pos, 10240, 256)[:N1, :32]
    return _final_chain(i1, h0, params)
